# baseline (speedup 1.0000x reference)
import functools

import jax
import jax.numpy as jnp
from jax import lax
from jax.experimental import pallas as pl
from jax.experimental.pallas import tpu as pltpu
from jax.experimental.pallas import tpu_sc as plsc

_NUM_EMB = 1000000
_DIM = 64
_BATCH = 16384
_HIST = 50
_N_IDX = _BATCH * _HIST

_MM_BLOCK = 8000

_ROUND_MAGIC = 12582912.0


_RP_W = 24960
_RP_NFULL = _NUM_EMB // _RP_W
_RP_TAIL = _NUM_EMB - _RP_NFULL * _RP_W
_RP_E = _RP_W * _DIM
_RP_TE = _RP_TAIL * _DIM


def _repack_body(w_hbm, wtail_hbm, lin_hbm, mm_ref,
                 inb, outb, tailb, acc_ref, sin, sout, stail):
    j = pl.program_id(0)
    jm = j % 2

    @pl.when(j == 0)
    def _prime():
        pltpu.make_async_copy(
            w_hbm.at[:, pl.ds(0, _RP_W)], inb.at[0], sin.at[0]).start()

    @pl.when(j + 1 < _RP_NFULL)
    def _prefetch():
        pltpu.make_async_copy(
            w_hbm.at[:, pl.ds((j + 1) * _RP_W, _RP_W)],
            inb.at[(j + 1) % 2], sin.at[(j + 1) % 2]).start()

    @pl.when(j < _RP_NFULL)
    def _main():
        pltpu.make_async_copy(
            w_hbm.at[:, pl.ds(j * _RP_W, _RP_W)], inb.at[jm], sin.at[jm]).wait()
        blk = inb[jm]
        bmin = jnp.min(blk)
        bmax = jnp.max(blk)

        @pl.when(j == 0)
        def _init():
            acc_ref[0] = bmin
            acc_ref[1] = bmax

        @pl.when(j > 0)
        def _acc():
            acc_ref[0] = jnp.minimum(acc_ref[0], bmin)
            acc_ref[1] = jnp.maximum(acc_ref[1], bmax)

        @pl.when(j >= 2)
        def _drain_out():
            pltpu.make_async_copy(
                outb.at[jm], lin_hbm.at[pl.ds((j - 2) * _RP_W, _RP_W), :],
                sout.at[jm]).wait()

        outb[jm, :, pl.ds(0, _DIM)] = blk.T
        pltpu.make_async_copy(
            outb.at[jm], lin_hbm.at[pl.ds(j * _RP_W, _RP_W), :],
            sout.at[jm]).start()

    @pl.when(j == _RP_NFULL)
    def _tail():
        pltpu.make_async_copy(wtail_hbm, tailb, stail).start()
        pltpu.make_async_copy(wtail_hbm, tailb, stail).wait()
        tb = tailb[...]
        acc_ref[0] = jnp.minimum(acc_ref[0], jnp.min(tb))
        acc_ref[1] = jnp.maximum(acc_ref[1], jnp.max(tb))
        pltpu.make_async_copy(
            outb.at[0], lin_hbm.at[pl.ds((_RP_NFULL - 2) * _RP_W, _RP_W), :],
            sout.at[0]).wait()
        outb[0, pl.ds(0, _RP_TAIL), pl.ds(0, _DIM)] = tb.T
        pltpu.make_async_copy(
            outb.at[0, pl.ds(0, _RP_TAIL), :],
            lin_hbm.at[pl.ds(_RP_NFULL * _RP_W, _RP_TAIL), :], sout.at[0]).start()
        pltpu.make_async_copy(
            outb.at[1], lin_hbm.at[pl.ds((_RP_NFULL - 1) * _RP_W, _RP_W), :],
            sout.at[1]).wait()
        pltpu.make_async_copy(
            outb.at[0, pl.ds(0, _RP_TAIL), :],
            lin_hbm.at[pl.ds(_RP_NFULL * _RP_W, _RP_TAIL), :], sout.at[0]).wait()
        mm_ref[0] = acc_ref[0]
        mm_ref[1] = acc_ref[1]


def _repack_minmax(weight_t, weight_tail):
    return pl.pallas_call(
        _repack_body,
        grid=(_RP_NFULL + 1,),
        in_specs=[pl.BlockSpec(memory_space=pl.ANY),
                  pl.BlockSpec(memory_space=pl.ANY)],
        out_specs=[pl.BlockSpec(memory_space=pl.ANY),
                   pl.BlockSpec(memory_space=pltpu.SMEM)],
        out_shape=[jax.ShapeDtypeStruct((_NUM_EMB, 128), jnp.float32),
                   jax.ShapeDtypeStruct((2,), jnp.float32)],
        scratch_shapes=[
            pltpu.VMEM((2, _DIM, _RP_W), jnp.float32),
            pltpu.VMEM((2, _RP_W, 128), jnp.float32),
            pltpu.VMEM((_DIM, _RP_TAIL), jnp.float32),
            pltpu.SMEM((2,), jnp.float32),
            pltpu.SemaphoreType.DMA((2,)),
            pltpu.SemaphoreType.DMA((2,)),
            pltpu.SemaphoreType.DMA,
        ],
        compiler_params=pltpu.CompilerParams(vmem_limit_bytes=110_000_000),
    )(weight_t, weight_tail)


def _make_gather_kernel(n_workers, b_per_w, chunk, n_out=_N_IDX):
    n_chunks = b_per_w // chunk
    n_pairs = n_chunks // 2
    mesh = plsc.VectorSubcoreMesh(core_axis_name="c", subcore_axis_name="s")

    scratch_types = [
            pltpu.VMEM((2, chunk), jnp.int32),
            pltpu.VMEM((chunk, 128), jnp.float32),
            pltpu.VMEM((chunk, 128), jnp.float32),
            pltpu.VMEM((4, 16), jnp.float32),
            pltpu.SemaphoreType.DMA,
            pltpu.SemaphoreType.DMA,
            pltpu.SemaphoreType.DMA,
            pltpu.SemaphoreType.DMA,
        ]

    @functools.partial(
        pl.kernel,
        mesh=mesh,
        compiler_params=pltpu.CompilerParams(use_tc_tiling_on_sc=False),
        out_type=jax.ShapeDtypeStruct((n_out, _DIM), jnp.float32),
        scratch_types=scratch_types,
    )
    def gather_dequant(idx_hbm, table_hbm, params_hbm, out_hbm,
                       idx_v, rows0, rows1, params_v, sg0, sg1, ss0, ss1):
        wid = lax.axis_index("s") * 2 + lax.axis_index("c")
        base = wid * b_per_w
        pltpu.sync_copy(params_hbm, params_v)
        inv_scale = params_v[0, :]
        zp = params_v[1, :]
        scale = params_v[2, :]
        zp_scale = params_v[3, :]

        def dequant(buf):
            def row_body(r, c2):
                for j in range(_DIM // 16):
                    w = buf[r, pl.ds(j * 16, 16)]
                    t = w * inv_scale + zp
                    t = jnp.maximum(t, -128.0)
                    t = jnp.minimum(t, 127.0)
                    q = (t + _ROUND_MAGIC) - _ROUND_MAGIC
                    buf[r, pl.ds(j * 16, 16)] = q * scale - zp_scale
                return c2
            lax.fori_loop(0, chunk, row_body, 0)

        def drain_gather(buf, sem):
            pltpu.make_async_copy(
                table_hbm.at[pl.ds(0, chunk), :], buf, sem).wait()

        def drain_store(buf, sem):
            pltpu.make_async_copy(
                buf.at[:, pl.ds(0, _DIM)],
                out_hbm.at[pl.ds(base, chunk)], sem).wait()

        pltpu.sync_copy(idx_hbm.at[pl.ds(base, chunk)], idx_v.at[0])
        pltpu.async_copy(table_hbm.at[idx_v.at[0]], rows0, sg0)

        def pair_body(p, carry):
            k0 = 2 * p
            off0 = base + k0 * chunk
            off1 = off0 + chunk

            @pl.when(p > 0)
            def _():
                drain_store(rows1, ss1)

            pltpu.sync_copy(idx_hbm.at[pl.ds(off1, chunk)], idx_v.at[1])
            pltpu.async_copy(table_hbm.at[idx_v.at[1]], rows1, sg1)

            drain_gather(rows0, sg0)
            dequant(rows0)
            pltpu.async_copy(rows0.at[:, pl.ds(0, _DIM)],
                             out_hbm.at[pl.ds(off0, chunk)], ss0)

            @pl.when(p < n_pairs - 1)
            def _():
                drain_store(rows0, ss0)
                pltpu.sync_copy(
                    idx_hbm.at[pl.ds(off0 + 2 * chunk, chunk)], idx_v.at[0])
                pltpu.async_copy(table_hbm.at[idx_v.at[0]], rows0, sg0)

            drain_gather(rows1, sg1)
            dequant(rows1)
            pltpu.async_copy(rows1.at[:, pl.ds(0, _DIM)],
                             out_hbm.at[pl.ds(off1, chunk)], ss1)
            return carry

        lax.fori_loop(0, n_pairs, pair_body, 0)
        drain_store(rows0, ss0)
        drain_store(rows1, ss1)

    return gather_dequant


def kernel(x, weight):
    wt = weight.T
    table_lin, mm = _repack_minmax(wt, wt[:, _RP_NFULL * _RP_W:])
    wmin, wmax = mm[0], mm[1]
    scale = (wmax - wmin) / 255.0
    zp = -128.0 - wmin / scale
    params = jnp.stack([
        jnp.full((16,), 1.0 / scale, jnp.float32),
        jnp.full((16,), zp, jnp.float32),
        jnp.full((16,), scale, jnp.float32),
        jnp.full((16,), zp * scale, jnp.float32),
    ])

    info = plsc.get_sparse_core_info()
    n_workers = info.num_cores * info.num_subcores
    nsplit = 4
    n_part = _N_IDX // nsplit
    b_per_w = n_part // n_workers
    k = _make_gather_kernel(n_workers, b_per_w, chunk=400, n_out=n_part)
    xf = x.reshape(-1)
    out = jnp.zeros((_BATCH, _HIST, _DIM), jnp.float32)
    for i in range(nsplit):
        o = k(xf[i * n_part:(i + 1) * n_part], table_lin, params)
        out = lax.dynamic_update_slice(
            out, o.reshape(_BATCH // nsplit, _HIST, _DIM),
            (i * (_BATCH // nsplit), 0, 0))
    return out


# speedup vs baseline: 2.2441x; 2.2441x over previous
import functools

import jax
import jax.numpy as jnp
from jax import lax
from jax.experimental import pallas as pl
from jax.experimental.pallas import tpu as pltpu
from jax.experimental.pallas import tpu_sc as plsc

_NUM_EMB = 1000000
_DIM = 64
_BATCH = 16384
_HIST = 50
_N_IDX = _BATCH * _HIST

_MM_BLOCK = 8000

_ROUND_MAGIC = 12582912.0


_RP_W = 24960
_RP_NFULL = _NUM_EMB // _RP_W
_RP_TAIL = _NUM_EMB - _RP_NFULL * _RP_W
_RP_E = _RP_W * _DIM
_RP_TE = _RP_TAIL * _DIM


def _repack_body(w_hbm, wtail_hbm, lin_hbm, mm_ref,
                 inb, outb, tailb, acc_ref, sin, sout, stail):
    j = pl.program_id(0)
    jm = j % 2

    @pl.when(j == 0)
    def _prime():
        pltpu.make_async_copy(
            w_hbm.at[:, pl.ds(0, _RP_W)], inb.at[0], sin.at[0]).start()

    @pl.when(j + 1 < _RP_NFULL)
    def _prefetch():
        pltpu.make_async_copy(
            w_hbm.at[:, pl.ds((j + 1) * _RP_W, _RP_W)],
            inb.at[(j + 1) % 2], sin.at[(j + 1) % 2]).start()

    @pl.when(j < _RP_NFULL)
    def _main():
        pltpu.make_async_copy(
            w_hbm.at[:, pl.ds(j * _RP_W, _RP_W)], inb.at[jm], sin.at[jm]).wait()
        blk = inb[jm]
        bmin = jnp.min(blk)
        bmax = jnp.max(blk)

        @pl.when(j == 0)
        def _init():
            acc_ref[0] = bmin
            acc_ref[1] = bmax

        @pl.when(j > 0)
        def _acc():
            acc_ref[0] = jnp.minimum(acc_ref[0], bmin)
            acc_ref[1] = jnp.maximum(acc_ref[1], bmax)

        @pl.when(j >= 2)
        def _drain_out():
            pltpu.make_async_copy(
                outb.at[jm], lin_hbm.at[pl.ds((j - 2) * _RP_W, _RP_W), :],
                sout.at[jm]).wait()

        outb[jm, :, pl.ds(0, _DIM)] = blk.T
        pltpu.make_async_copy(
            outb.at[jm], lin_hbm.at[pl.ds(j * _RP_W, _RP_W), :],
            sout.at[jm]).start()

    @pl.when(j == _RP_NFULL)
    def _tail():
        pltpu.make_async_copy(wtail_hbm, tailb, stail).start()
        pltpu.make_async_copy(wtail_hbm, tailb, stail).wait()
        tb = tailb[...]
        acc_ref[0] = jnp.minimum(acc_ref[0], jnp.min(tb))
        acc_ref[1] = jnp.maximum(acc_ref[1], jnp.max(tb))
        pltpu.make_async_copy(
            outb.at[0], lin_hbm.at[pl.ds((_RP_NFULL - 2) * _RP_W, _RP_W), :],
            sout.at[0]).wait()
        outb[0, pl.ds(0, _RP_TAIL), pl.ds(0, _DIM)] = tb.T
        pltpu.make_async_copy(
            outb.at[0, pl.ds(0, _RP_TAIL), :],
            lin_hbm.at[pl.ds(_RP_NFULL * _RP_W, _RP_TAIL), :], sout.at[0]).start()
        pltpu.make_async_copy(
            outb.at[1], lin_hbm.at[pl.ds((_RP_NFULL - 1) * _RP_W, _RP_W), :],
            sout.at[1]).wait()
        pltpu.make_async_copy(
            outb.at[0, pl.ds(0, _RP_TAIL), :],
            lin_hbm.at[pl.ds(_RP_NFULL * _RP_W, _RP_TAIL), :], sout.at[0]).wait()
        mm_ref[0] = acc_ref[0]
        mm_ref[1] = acc_ref[1]


def _repack_minmax(weight_t, weight_tail):
    return pl.pallas_call(
        _repack_body,
        grid=(_RP_NFULL + 1,),
        in_specs=[pl.BlockSpec(memory_space=pl.ANY),
                  pl.BlockSpec(memory_space=pl.ANY)],
        out_specs=[pl.BlockSpec(memory_space=pl.ANY),
                   pl.BlockSpec(memory_space=pltpu.SMEM)],
        out_shape=[jax.ShapeDtypeStruct((_NUM_EMB, 128), jnp.float32),
                   jax.ShapeDtypeStruct((2,), jnp.float32)],
        scratch_shapes=[
            pltpu.VMEM((2, _DIM, _RP_W), jnp.float32),
            pltpu.VMEM((2, _RP_W, 128), jnp.float32),
            pltpu.VMEM((_DIM, _RP_TAIL), jnp.float32),
            pltpu.SMEM((2,), jnp.float32),
            pltpu.SemaphoreType.DMA((2,)),
            pltpu.SemaphoreType.DMA((2,)),
            pltpu.SemaphoreType.DMA,
        ],
        compiler_params=pltpu.CompilerParams(vmem_limit_bytes=110_000_000),
    )(weight_t, weight_tail)


def _make_gather_kernel(n_workers, b_per_w, chunk, n_out=_N_IDX):
    n_chunks = b_per_w // chunk
    n_pairs = n_chunks // 2
    mesh = plsc.VectorSubcoreMesh(core_axis_name="c", subcore_axis_name="s")

    scratch_types = [
            pltpu.VMEM((2, chunk), jnp.int32),
            pltpu.VMEM((chunk, 128), jnp.float32),
            pltpu.VMEM((chunk, 128), jnp.float32),
            pltpu.VMEM((4, 16), jnp.float32),
            pltpu.SemaphoreType.DMA,
            pltpu.SemaphoreType.DMA,
            pltpu.SemaphoreType.DMA,
            pltpu.SemaphoreType.DMA,
        ]

    @functools.partial(
        pl.kernel,
        mesh=mesh,
        compiler_params=pltpu.CompilerParams(use_tc_tiling_on_sc=False),
        out_type=jax.ShapeDtypeStruct((n_out, _DIM), jnp.float32),
        scratch_types=scratch_types,
    )
    def gather_dequant(idx_hbm, table_hbm, params_hbm, out_hbm,
                       idx_v, rows0, rows1, params_v, sg0, sg1, ss0, ss1):
        wid = lax.axis_index("s") * 2 + lax.axis_index("c")
        base = wid * b_per_w
        pltpu.sync_copy(params_hbm, params_v)
        inv_scale = params_v[0, :]
        zp = params_v[1, :]
        scale = params_v[2, :]
        zp_scale = params_v[3, :]

        def dequant(buf):
            def row_body(r, c2):
                for j in range(_DIM // 16):
                    w = buf[r, pl.ds(j * 16, 16)]
                    t = w * inv_scale + zp
                    t = jnp.maximum(t, -128.0)
                    t = jnp.minimum(t, 127.0)
                    q = (t + _ROUND_MAGIC) - _ROUND_MAGIC
                    buf[r, pl.ds(j * 16, 16)] = q * scale - zp_scale
                return c2
            lax.fori_loop(0, chunk, row_body, 0)

        def drain_gather(buf, sem):
            pltpu.make_async_copy(
                table_hbm.at[pl.ds(0, chunk), :], buf, sem).wait()

        def drain_store(buf, sem):
            pltpu.make_async_copy(
                buf.at[:, pl.ds(0, _DIM)],
                out_hbm.at[pl.ds(base, chunk)], sem).wait()

        pltpu.sync_copy(idx_hbm.at[pl.ds(base, chunk)], idx_v.at[0])
        pltpu.async_copy(table_hbm.at[idx_v.at[0]], rows0, sg0)

        def pair_body(p, carry):
            k0 = 2 * p
            off0 = base + k0 * chunk
            off1 = off0 + chunk

            @pl.when(p > 0)
            def _():
                drain_store(rows1, ss1)

            pltpu.sync_copy(idx_hbm.at[pl.ds(off1, chunk)], idx_v.at[1])
            pltpu.async_copy(table_hbm.at[idx_v.at[1]], rows1, sg1)

            drain_gather(rows0, sg0)
            dequant(rows0)
            pltpu.async_copy(rows0.at[:, pl.ds(0, _DIM)],
                             out_hbm.at[pl.ds(off0, chunk)], ss0)

            @pl.when(p < n_pairs - 1)
            def _():
                drain_store(rows0, ss0)
                pltpu.sync_copy(
                    idx_hbm.at[pl.ds(off0 + 2 * chunk, chunk)], idx_v.at[0])
                pltpu.async_copy(table_hbm.at[idx_v.at[0]], rows0, sg0)

            drain_gather(rows1, sg1)
            dequant(rows1)
            pltpu.async_copy(rows1.at[:, pl.ds(0, _DIM)],
                             out_hbm.at[pl.ds(off1, chunk)], ss1)
            return carry

        lax.fori_loop(0, n_pairs, pair_body, 0)
        drain_store(rows0, ss0)
        drain_store(rows1, ss1)

    return gather_dequant


def kernel(x, weight):
    wt = weight.T
    table_lin, mm = _repack_minmax(wt, wt[:, _RP_NFULL * _RP_W:])
    wmin, wmax = mm[0], mm[1]
    scale = (wmax - wmin) / 255.0
    zp = -128.0 - wmin / scale
    params = jnp.stack([
        jnp.full((16,), 1.0 / scale, jnp.float32),
        jnp.full((16,), zp, jnp.float32),
        jnp.full((16,), scale, jnp.float32),
        jnp.full((16,), zp * scale, jnp.float32),
    ])

    info = plsc.get_sparse_core_info()
    n_workers = info.num_cores * info.num_subcores
    nsplit = 8
    n_part = _N_IDX // nsplit
    b_per_w = n_part // n_workers
    k = _make_gather_kernel(n_workers, b_per_w, chunk=400, n_out=n_part)
    xf = x.reshape(-1)
    parts = []
    for i in range(nsplit):
        o = k(xf[i * n_part:(i + 1) * n_part], table_lin, params)
        parts.append(o.reshape(_BATCH // nsplit, _HIST, _DIM))
    return jnp.concatenate(parts, axis=0)


# speedup vs baseline: 2.2726x; 1.0127x over previous
import functools

import jax
import jax.numpy as jnp
from jax import lax
from jax.experimental import pallas as pl
from jax.experimental.pallas import tpu as pltpu
from jax.experimental.pallas import tpu_sc as plsc

_NUM_EMB = 1000000
_DIM = 64
_BATCH = 16384
_HIST = 50
_N_IDX = _BATCH * _HIST

_MM_BLOCK = 8000

_ROUND_MAGIC = 12582912.0


_RP_W = 24960
_RP_NFULL = _NUM_EMB // _RP_W
_RP_TAIL = _NUM_EMB - _RP_NFULL * _RP_W
_RP_E = _RP_W * _DIM
_RP_TE = _RP_TAIL * _DIM


def _repack_body(w_hbm, wtail_hbm, lin_hbm, mm_ref,
                 inb, outb, tailb, acc_ref, sin, sout, stail):
    j = pl.program_id(0)
    jm = j % 2

    @pl.when(j == 0)
    def _prime():
        pltpu.make_async_copy(
            w_hbm.at[:, pl.ds(0, _RP_W)], inb.at[0], sin.at[0]).start()

    @pl.when(j + 1 < _RP_NFULL)
    def _prefetch():
        pltpu.make_async_copy(
            w_hbm.at[:, pl.ds((j + 1) * _RP_W, _RP_W)],
            inb.at[(j + 1) % 2], sin.at[(j + 1) % 2]).start()

    @pl.when(j < _RP_NFULL)
    def _main():
        pltpu.make_async_copy(
            w_hbm.at[:, pl.ds(j * _RP_W, _RP_W)], inb.at[jm], sin.at[jm]).wait()
        blk = inb[jm]
        bmin = jnp.min(blk)
        bmax = jnp.max(blk)

        @pl.when(j == 0)
        def _init():
            acc_ref[0] = bmin
            acc_ref[1] = bmax

        @pl.when(j > 0)
        def _acc():
            acc_ref[0] = jnp.minimum(acc_ref[0], bmin)
            acc_ref[1] = jnp.maximum(acc_ref[1], bmax)

        @pl.when(j >= 2)
        def _drain_out():
            pltpu.make_async_copy(
                outb.at[jm], lin_hbm.at[pl.ds((j - 2) * _RP_W, _RP_W), :],
                sout.at[jm]).wait()

        outb[jm, :, pl.ds(0, _DIM)] = blk.T
        pltpu.make_async_copy(
            outb.at[jm], lin_hbm.at[pl.ds(j * _RP_W, _RP_W), :],
            sout.at[jm]).start()

    @pl.when(j == _RP_NFULL)
    def _tail():
        pltpu.make_async_copy(wtail_hbm, tailb, stail).start()
        pltpu.make_async_copy(wtail_hbm, tailb, stail).wait()
        tb = tailb[...]
        acc_ref[0] = jnp.minimum(acc_ref[0], jnp.min(tb))
        acc_ref[1] = jnp.maximum(acc_ref[1], jnp.max(tb))
        pltpu.make_async_copy(
            outb.at[0], lin_hbm.at[pl.ds((_RP_NFULL - 2) * _RP_W, _RP_W), :],
            sout.at[0]).wait()
        outb[0, pl.ds(0, _RP_TAIL), pl.ds(0, _DIM)] = tb.T
        pltpu.make_async_copy(
            outb.at[0, pl.ds(0, _RP_TAIL), :],
            lin_hbm.at[pl.ds(_RP_NFULL * _RP_W, _RP_TAIL), :], sout.at[0]).start()
        pltpu.make_async_copy(
            outb.at[1], lin_hbm.at[pl.ds((_RP_NFULL - 1) * _RP_W, _RP_W), :],
            sout.at[1]).wait()
        pltpu.make_async_copy(
            outb.at[0, pl.ds(0, _RP_TAIL), :],
            lin_hbm.at[pl.ds(_RP_NFULL * _RP_W, _RP_TAIL), :], sout.at[0]).wait()
        mm_ref[0] = acc_ref[0]
        mm_ref[1] = acc_ref[1]


def _repack_minmax(weight_t, weight_tail):
    return pl.pallas_call(
        _repack_body,
        grid=(_RP_NFULL + 1,),
        in_specs=[pl.BlockSpec(memory_space=pl.ANY),
                  pl.BlockSpec(memory_space=pl.ANY)],
        out_specs=[pl.BlockSpec(memory_space=pl.ANY),
                   pl.BlockSpec(memory_space=pltpu.SMEM)],
        out_shape=[jax.ShapeDtypeStruct((_NUM_EMB, 128), jnp.float32),
                   jax.ShapeDtypeStruct((2,), jnp.float32)],
        scratch_shapes=[
            pltpu.VMEM((2, _DIM, _RP_W), jnp.float32),
            pltpu.VMEM((2, _RP_W, 128), jnp.float32),
            pltpu.VMEM((_DIM, _RP_TAIL), jnp.float32),
            pltpu.SMEM((2,), jnp.float32),
            pltpu.SemaphoreType.DMA((2,)),
            pltpu.SemaphoreType.DMA((2,)),
            pltpu.SemaphoreType.DMA,
        ],
        compiler_params=pltpu.CompilerParams(vmem_limit_bytes=110_000_000),
    )(weight_t, weight_tail)


def _make_gather_kernel(n_workers, b_per_w, chunk, n_out=_N_IDX):
    n_chunks = b_per_w // chunk
    n_pairs = n_chunks // 2
    mesh = plsc.VectorSubcoreMesh(core_axis_name="c", subcore_axis_name="s")

    scratch_types = [
            pltpu.VMEM((2, chunk), jnp.int32),
            pltpu.VMEM((chunk, 128), jnp.float32),
            pltpu.VMEM((chunk, 128), jnp.float32),
            pltpu.VMEM((4, 16), jnp.float32),
            pltpu.SemaphoreType.DMA,
            pltpu.SemaphoreType.DMA,
            pltpu.SemaphoreType.DMA,
            pltpu.SemaphoreType.DMA,
        ]

    @functools.partial(
        pl.kernel,
        mesh=mesh,
        compiler_params=pltpu.CompilerParams(use_tc_tiling_on_sc=False),
        out_type=jax.ShapeDtypeStruct((n_out, _DIM), jnp.float32),
        scratch_types=scratch_types,
    )
    def gather_dequant(idx_hbm, table_hbm, params_hbm, out_hbm,
                       idx_v, rows0, rows1, params_v, sg0, sg1, ss0, ss1):
        wid = lax.axis_index("s") * 2 + lax.axis_index("c")
        base = wid * b_per_w
        pltpu.sync_copy(params_hbm, params_v)
        inv_scale = params_v[0, :]
        zp = params_v[1, :]
        scale = params_v[2, :]
        zp_scale = params_v[3, :]

        def dequant(buf):
            def row_body(r, c2):
                for j in range(_DIM // 16):
                    w = buf[r, pl.ds(j * 16, 16)]
                    t = w * inv_scale + zp
                    t = jnp.maximum(t, -128.0)
                    t = jnp.minimum(t, 127.0)
                    q = (t + _ROUND_MAGIC) - _ROUND_MAGIC
                    buf[r, pl.ds(j * 16, 16)] = q * scale - zp_scale
                return c2
            lax.fori_loop(0, chunk, row_body, 0)

        def drain_gather(buf, sem):
            pltpu.make_async_copy(
                table_hbm.at[pl.ds(0, chunk), :], buf, sem).wait()

        def drain_store(buf, sem):
            pltpu.make_async_copy(
                buf.at[:, pl.ds(0, _DIM)],
                out_hbm.at[pl.ds(base, chunk)], sem).wait()

        pltpu.sync_copy(idx_hbm.at[pl.ds(base, chunk)], idx_v.at[0])
        pltpu.async_copy(table_hbm.at[idx_v.at[0]], rows0, sg0)

        def pair_body(p, carry):
            k0 = 2 * p
            off0 = base + k0 * chunk
            off1 = off0 + chunk

            @pl.when(p > 0)
            def _():
                drain_store(rows1, ss1)

            pltpu.sync_copy(idx_hbm.at[pl.ds(off1, chunk)], idx_v.at[1])
            pltpu.async_copy(table_hbm.at[idx_v.at[1]], rows1, sg1)

            drain_gather(rows0, sg0)
            dequant(rows0)
            pltpu.async_copy(rows0.at[:, pl.ds(0, _DIM)],
                             out_hbm.at[pl.ds(off0, chunk)], ss0)

            @pl.when(p < n_pairs - 1)
            def _():
                drain_store(rows0, ss0)
                pltpu.sync_copy(
                    idx_hbm.at[pl.ds(off0 + 2 * chunk, chunk)], idx_v.at[0])
                pltpu.async_copy(table_hbm.at[idx_v.at[0]], rows0, sg0)

            drain_gather(rows1, sg1)
            dequant(rows1)
            pltpu.async_copy(rows1.at[:, pl.ds(0, _DIM)],
                             out_hbm.at[pl.ds(off1, chunk)], ss1)
            return carry

        lax.fori_loop(0, n_pairs, pair_body, 0)
        drain_store(rows0, ss0)
        drain_store(rows1, ss1)

    return gather_dequant


def kernel(x, weight):
    wt = weight.T
    table_lin, mm = _repack_minmax(wt, wt[:, _RP_NFULL * _RP_W:])
    wmin, wmax = mm[0], mm[1]
    scale = (wmax - wmin) / 255.0
    zp = -128.0 - wmin / scale
    params = jnp.stack([
        jnp.full((16,), 1.0 / scale, jnp.float32),
        jnp.full((16,), zp, jnp.float32),
        jnp.full((16,), scale, jnp.float32),
        jnp.full((16,), zp * scale, jnp.float32),
    ])

    info = plsc.get_sparse_core_info()
    n_workers = info.num_cores * info.num_subcores
    nsplit = 4
    n_part = _N_IDX // nsplit
    b_per_w = n_part // n_workers
    k = _make_gather_kernel(n_workers, b_per_w, chunk=400, n_out=n_part)
    xf = x.reshape(-1)
    parts = []
    for i in range(nsplit):
        o = k(xf[i * n_part:(i + 1) * n_part], table_lin, params)
        parts.append(o.reshape(_BATCH // nsplit, _HIST, _DIM))
    return jnp.concatenate(parts, axis=0)
